# baseline (speedup 1.0000x reference)
import functools

import jax
import jax.numpy as jnp
from jax import lax
from jax.experimental import pallas as pl
from jax.experimental.pallas import tpu as pltpu
from jax.experimental.pallas import tpu_sc as plsc

_NC, _NS = 2, 16
_NW = _NC * _NS
_KC = 80


def _silu(x):
    return x * jax.nn.sigmoid(x)


def _p0_body(h_ref, w_ref, a_ref, b_ref, c_ref):
    hw = lax.dot_general(h_ref[...], w_ref[...], (((1,), (0,)), ((), ())),
                         preferred_element_type=jnp.float32)
    a_ref[...] = hw[:, 0:128]
    b_ref[...] = hw[:, 128:256]
    c_ref[...] = hw[:, 256:384]


def _precompute_tables(hf, Wcat):
    n, f = hf.shape
    blk = 2000
    grid = n // blk
    return pl.pallas_call(
        _p0_body,
        grid=(grid,),
        in_specs=[
            pl.BlockSpec((blk, f), lambda i: (i, 0)),
            pl.BlockSpec((f, 384), lambda i: (0, 0)),
        ],
        out_specs=[
            pl.BlockSpec((blk, 128), lambda i: (i, 0)),
            pl.BlockSpec((blk, 128), lambda i: (i, 0)),
            pl.BlockSpec((blk, 128), lambda i: (i, 0)),
        ],
        out_shape=[jax.ShapeDtypeStruct((n, 128), jnp.float32)] * 3,
    )(hf, Wcat)


def _sc_gather_add(A, Bm, row, col):
    n, f = A.shape
    e = row.shape[0]
    e_per_w = e // _NW
    nchunks = e_per_w // _KC
    mesh = plsc.VectorSubcoreMesh(core_axis_name="c", subcore_axis_name="s",
                                  num_cores=_NC, num_subcores=_NS)

    @functools.partial(
        pl.kernel, mesh=mesh,
        out_type=jax.ShapeDtypeStruct((e, f // 2), jnp.float32),
        scratch_types=[
            pltpu.VMEM((e_per_w,), jnp.int32),
            pltpu.VMEM((e_per_w,), jnp.int32),
            pltpu.VMEM((2, _KC, 128), jnp.float32),
            pltpu.VMEM((2, _KC, 128), jnp.float32),
            pltpu.VMEM((2, _KC, 64), jnp.float32),
            pltpu.SemaphoreType.DMA((2,)),
            pltpu.SemaphoreType.DMA((2,)),
        ],
    )
    def k(a_hbm, b_hbm, row_hbm, col_hbm, out_hbm,
          idxr_v, idxc_v, arows_v, brows_v, obuf_v, sem_a, sem_b):
        wid = lax.axis_index("s") * _NC + lax.axis_index("c")
        base = wid * e_per_w
        pltpu.sync_copy(row_hbm.at[pl.ds(base, e_per_w)], idxr_v)
        pltpu.sync_copy(col_hbm.at[pl.ds(base, e_per_w)], idxc_v)

        def fire(j, s):
            pltpu.async_copy(a_hbm.at[idxr_v.at[pl.ds(j * _KC, _KC)]],
                             arows_v.at[s], sem_a.at[s])
            pltpu.async_copy(b_hbm.at[idxc_v.at[pl.ds(j * _KC, _KC)]],
                             brows_v.at[s], sem_b.at[s])

        fire(0, 0)

        def chunk(j, carry):
            s = lax.rem(j, 2)

            @pl.when(j + 1 < nchunks)
            def _prefetch():
                fire(j + 1, 1 - s)

            pltpu.make_async_copy(a_hbm.at[idxr_v.at[pl.ds(j * _KC, _KC)]],
                                  arows_v.at[s], sem_a.at[s]).wait()
            pltpu.make_async_copy(b_hbm.at[idxc_v.at[pl.ds(j * _KC, _KC)]],
                                  brows_v.at[s], sem_b.at[s]).wait()

            def rnd16(x):
                u = lax.bitcast_convert_type(x, jnp.uint32)
                rnd = jnp.uint32(0x7FFF) + (
                    lax.shift_right_logical(u, jnp.uint32(16)) & jnp.uint32(1))
                return lax.shift_right_logical(u + rnd, jnp.uint32(16))

            def add_loop(b):
                def addrow(r, c2):
                    for g in range(4):
                        lo = pl.ds(g * 32, 16)
                        hi = pl.ds(g * 32 + 16, 16)
                        s_lo = arows_v[b, r, lo] + brows_v[b, r, lo]
                        s_hi = arows_v[b, r, hi] + brows_v[b, r, hi]
                        word = rnd16(s_lo) | lax.shift_left(
                            rnd16(s_hi), jnp.uint32(16))
                        obuf_v[b, r, pl.ds(g * 16, 16)] = (
                            lax.bitcast_convert_type(word, jnp.float32))
                    return c2
                lax.fori_loop(0, _KC, addrow, 0, unroll=2)

            @pl.when(s == 0)
            def _add0():
                add_loop(0)

            @pl.when(s == 1)
            def _add1():
                add_loop(1)

            pltpu.sync_copy(obuf_v.at[s],
                            out_hbm.at[pl.ds(base + j * _KC, _KC)])
            return carry

        lax.fori_loop(0, nchunks, chunk, 0)

    return k(A, Bm, row, col)


def _sc_scatter_add(mij, row, n):
    e, f = mij.shape
    e_per_w = e // _NW
    nchunks = e_per_w // _KC
    n_main = 624
    n_tail = n - _NS * n_main
    zrows = 104
    mesh = plsc.VectorSubcoreMesh(core_axis_name="c", subcore_axis_name="s",
                                  num_cores=_NC, num_subcores=_NS)

    @functools.partial(
        pl.kernel, mesh=mesh,
        out_type=jax.ShapeDtypeStruct((_NC * n, f), jnp.float32),
        scratch_types=[
            pltpu.VMEM((e_per_w,), jnp.int32),
            pltpu.VMEM((_KC,), jnp.int32),
            pltpu.VMEM((2, _KC, 128), jnp.float32),
            pltpu.VMEM((zrows, 128), jnp.float32),
            pltpu.VMEM_SHARED((n, 128), jnp.float32),
            pltpu.SemaphoreType.DMA((2,)),
        ],
    )
    def k(mij_hbm, row_hbm, out_hbm, idx_all, idx_sc, rows_v, zero_v,
          agg_sh, sem):
        cid = lax.axis_index("c")
        sid = lax.axis_index("s")
        wid = sid * _NC + cid
        base = wid * e_per_w
        pltpu.sync_copy(row_hbm.at[pl.ds(base, e_per_w)], idx_all)

        def zrow(r, c2):
            def zvec(v, c3):
                zero_v[r, pl.ds(v * 16, 16)] = jnp.zeros((16,), jnp.float32)
                return c3
            return lax.fori_loop(0, 8, zvec, c2)
        lax.fori_loop(0, zrows, zrow, 0)

        def zcp(b, c2):
            pltpu.sync_copy(zero_v,
                            agg_sh.at[pl.ds(sid * n_main + b * zrows, zrows)])
            return c2
        lax.fori_loop(0, n_main // zrows, zcp, 0)

        @pl.when(sid == _NS - 1)
        def _ztail():
            pltpu.sync_copy(zero_v.at[pl.ds(0, n_tail)],
                            agg_sh.at[pl.ds(_NS * n_main, n_tail)])
        plsc.subcore_barrier()

        def fire(j, s):
            pltpu.async_copy(mij_hbm.at[pl.ds(base + j * _KC, _KC)],
                             rows_v.at[s], sem.at[s])

        fire(0, 0)

        def chunk(j, carry):
            s = lax.rem(j, 2)

            @pl.when(j + 1 < nchunks)
            def _prefetch():
                fire(j + 1, 1 - s)

            def icp(v, c2):
                sl = pl.ds(v * 16, 16)
                idx_sc[sl] = idx_all[pl.ds(j * _KC + v * 16, 16)]
                return c2
            lax.fori_loop(0, _KC // 16, icp, 0)

            pltpu.make_async_copy(mij_hbm.at[pl.ds(base + j * _KC, _KC)],
                                  rows_v.at[s], sem.at[s]).wait()
            pltpu.sync_copy(rows_v.at[s], agg_sh.at[idx_sc], add=True)
            return carry
        lax.fori_loop(0, nchunks, chunk, 0)
        plsc.subcore_barrier()

        pltpu.sync_copy(agg_sh.at[pl.ds(sid * n_main, n_main)],
                        out_hbm.at[pl.ds(cid * n + sid * n_main, n_main)])

        @pl.when(sid == _NS - 1)
        def _otail():
            pltpu.sync_copy(agg_sh.at[pl.ds(_NS * n_main, n_tail)],
                            out_hbm.at[pl.ds(cid * n + _NS * n_main, n_tail)])

    res = k(mij, row)
    return res.reshape(_NC, n, f)


def _p2_body(gath_ref, ea_ref, mask_ref, w1e_ref, b1_ref, w2_ref, b2_ref,
             mij_ref):
    u = lax.bitcast_convert_type(gath_ref[...], jnp.uint32)
    f_even = lax.bitcast_convert_type(u << 16, jnp.float32)
    f_odd = lax.bitcast_convert_type(u & jnp.uint32(0xFFFF0000), jnp.float32)
    gath = jnp.concatenate([f_even, f_odd], axis=1)
    pre1 = mask_ref[...] * gath
    pre1 += lax.dot_general(ea_ref[...], w1e_ref[...], (((1,), (0,)), ((), ())),
                            preferred_element_type=jnp.float32)
    pre1 += b1_ref[...]
    h1 = _silu(pre1)
    h2 = lax.dot_general(h1, w2_ref[...], (((1,), (0,)), ((), ())),
                         preferred_element_type=jnp.float32) + b2_ref[...]
    mij_ref[...] = _silu(h2)


def _edge_mlp(gath, ea, mask, We1e, be1, We2, be2):
    e = gath.shape[0]
    blk = 2000
    grid = e // blk
    return pl.pallas_call(
        _p2_body,
        grid=(grid,),
        in_specs=[
            pl.BlockSpec((blk, 64), lambda i: (i, 0)),
            pl.BlockSpec((blk, 16), lambda i: (i, 0)),
            pl.BlockSpec((blk, 1), lambda i: (i, 0)),
            pl.BlockSpec((16, 128), lambda i: (0, 0)),
            pl.BlockSpec((1, 128), lambda i: (0, 0)),
            pl.BlockSpec((128, 128), lambda i: (0, 0)),
            pl.BlockSpec((1, 128), lambda i: (0, 0)),
        ],
        out_specs=pl.BlockSpec((blk, 128), lambda i: (i, 0)),
        out_shape=jax.ShapeDtypeStruct((e, 128), jnp.float32),
    )(gath, ea, mask, We1e, be1.reshape(1, 128), We2, be2.reshape(1, 128))


def _p4_body(h_ref, c_ref, p0_ref, p1_ref, w1b_ref, b1_ref, w2_ref, b2_ref,
             out_ref):
    pre = c_ref[...] + b1_ref[...]
    agg = p0_ref[...] + p1_ref[...]
    pre += lax.dot_general(agg, w1b_ref[...], (((1,), (0,)), ((), ())),
                           preferred_element_type=jnp.float32)
    h1 = _silu(pre)
    o = lax.dot_general(h1, w2_ref[...], (((1,), (0,)), ((), ())),
                        preferred_element_type=jnp.float32) + b2_ref[...]
    out_ref[...] = h_ref[...] + o


def _node_mlp(hf, C, p0, p1, Wn1b, bn1, Wn2, bn2):
    n, f = hf.shape
    blk = 2000
    grid = n // blk
    return pl.pallas_call(
        _p4_body,
        grid=(grid,),
        in_specs=[
            pl.BlockSpec((blk, f), lambda i: (i, 0)),
            pl.BlockSpec((blk, 128), lambda i: (i, 0)),
            pl.BlockSpec((blk, 128), lambda i: (i, 0)),
            pl.BlockSpec((blk, 128), lambda i: (i, 0)),
            pl.BlockSpec((128, 128), lambda i: (0, 0)),
            pl.BlockSpec((1, 128), lambda i: (0, 0)),
            pl.BlockSpec((128, f), lambda i: (0, 0)),
            pl.BlockSpec((1, f), lambda i: (0, 0)),
        ],
        out_specs=pl.BlockSpec((blk, f), lambda i: (i, 0)),
        out_shape=jax.ShapeDtypeStruct((n, f), jnp.float32),
    )(hf, C, p0, p1, Wn1b, bn1.reshape(1, 128), Wn2, bn2.reshape(1, f))


def kernel(h, edge_index, edge_attr, edge_mask, We1, be1, We2, be2,
           Wn1, bn1, Wn2, bn2):
    B, N, F = h.shape
    E = edge_index.shape[1]
    hf = h.reshape(N, F)
    row = edge_index[0, :, 0].astype(jnp.int32)
    col = edge_index[0, :, 1].astype(jnp.int32)
    ea = edge_attr.reshape(E, -1)
    mask = edge_mask.reshape(E, 1)

    Wcat = jnp.concatenate([We1[:F], We1[F:2 * F], Wn1[:F]], axis=1)
    A, Bm, C = _precompute_tables(hf, Wcat)

    gath = _sc_gather_add(A, Bm, row, col)

    k64 = jnp.arange(64)
    perm = jnp.concatenate([32 * (k64 // 16) + k64 % 16,
                            32 * (k64 // 16) + 16 + k64 % 16])
    mij = _edge_mlp(gath, ea, mask, We1[2 * F:][:, perm], be1[perm],
                    We2[perm, :], be2)

    partials = _sc_scatter_add(mij, row, N)

    out = _node_mlp(hf, C, partials[0], partials[1], Wn1[F:], bn1, Wn2, bn2)
    return (out.reshape(B, N, F), mij.reshape(B, E, 128))


# speedup vs baseline: 1.0618x; 1.0618x over previous
import functools

import jax
import jax.numpy as jnp
from jax import lax
from jax.experimental import pallas as pl
from jax.experimental.pallas import tpu as pltpu
from jax.experimental.pallas import tpu_sc as plsc

_NC, _NS = 2, 16
_NW = _NC * _NS
_KC = 80


def _silu(x):
    return x * jax.nn.sigmoid(x)


def _p0_body(h_ref, w_ref, a_ref, b_ref, c_ref):
    hw = lax.dot_general(h_ref[...], w_ref[...], (((1,), (0,)), ((), ())),
                         preferred_element_type=jnp.float32)
    a_ref[...] = hw[:, 0:128]
    b_ref[...] = hw[:, 128:256]
    c_ref[...] = hw[:, 256:384]


def _precompute_tables(hf, Wcat):
    n, f = hf.shape
    blk = 2000
    grid = n // blk
    return pl.pallas_call(
        _p0_body,
        grid=(grid,),
        in_specs=[
            pl.BlockSpec((blk, f), lambda i: (i, 0)),
            pl.BlockSpec((f, 384), lambda i: (0, 0)),
        ],
        out_specs=[
            pl.BlockSpec((blk, 128), lambda i: (i, 0)),
            pl.BlockSpec((blk, 128), lambda i: (i, 0)),
            pl.BlockSpec((blk, 128), lambda i: (i, 0)),
        ],
        out_shape=[jax.ShapeDtypeStruct((n, 128), jnp.float32)] * 3,
    )(hf, Wcat)


def _sc_gather_add(A, Bm, row, col):
    n, f = A.shape
    e = row.shape[0]
    e_per_w = e // _NW
    nchunks = e_per_w // _KC
    mesh = plsc.VectorSubcoreMesh(core_axis_name="c", subcore_axis_name="s",
                                  num_cores=_NC, num_subcores=_NS)

    @functools.partial(
        pl.kernel, mesh=mesh,
        out_type=jax.ShapeDtypeStruct((e, f // 2), jnp.float32),
        scratch_types=[
            pltpu.VMEM((e_per_w,), jnp.int32),
            pltpu.VMEM((e_per_w,), jnp.int32),
            pltpu.VMEM((2, _KC, 128), jnp.float32),
            pltpu.VMEM((2, _KC, 128), jnp.float32),
            pltpu.VMEM((2, _KC, 64), jnp.float32),
            pltpu.SemaphoreType.DMA((2,)),
            pltpu.SemaphoreType.DMA((2,)),
            pltpu.SemaphoreType.DMA((2,)),
        ],
    )
    def k(a_hbm, b_hbm, row_hbm, col_hbm, out_hbm,
          idxr_v, idxc_v, arows_v, brows_v, obuf_v, sem_a, sem_b, sem_w):
        wid = lax.axis_index("s") * _NC + lax.axis_index("c")
        base = wid * e_per_w
        pltpu.sync_copy(row_hbm.at[pl.ds(base, e_per_w)], idxr_v)
        pltpu.sync_copy(col_hbm.at[pl.ds(base, e_per_w)], idxc_v)

        def fire(j, s):
            pltpu.async_copy(a_hbm.at[idxr_v.at[pl.ds(j * _KC, _KC)]],
                             arows_v.at[s], sem_a.at[s])
            pltpu.async_copy(b_hbm.at[idxc_v.at[pl.ds(j * _KC, _KC)]],
                             brows_v.at[s], sem_b.at[s])

        fire(0, 0)

        def chunk(j, carry):
            s = lax.rem(j, 2)

            @pl.when(j >= 2)
            def _drain_wb():
                pltpu.make_async_copy(
                    obuf_v.at[s], out_hbm.at[pl.ds(base + (j - 2) * _KC, _KC)],
                    sem_w.at[s]).wait()

            @pl.when(j + 1 < nchunks)
            def _prefetch():
                fire(j + 1, 1 - s)

            pltpu.make_async_copy(a_hbm.at[idxr_v.at[pl.ds(j * _KC, _KC)]],
                                  arows_v.at[s], sem_a.at[s]).wait()
            pltpu.make_async_copy(b_hbm.at[idxc_v.at[pl.ds(j * _KC, _KC)]],
                                  brows_v.at[s], sem_b.at[s]).wait()

            def rnd16(x):
                u = lax.bitcast_convert_type(x, jnp.uint32)
                rnd = jnp.uint32(0x7FFF) + (
                    lax.shift_right_logical(u, jnp.uint32(16)) & jnp.uint32(1))
                return lax.shift_right_logical(u + rnd, jnp.uint32(16))

            def add_loop(b):
                def addrow(r, c2):
                    for g in range(4):
                        lo = pl.ds(g * 32, 16)
                        hi = pl.ds(g * 32 + 16, 16)
                        s_lo = arows_v[b, r, lo] + brows_v[b, r, lo]
                        s_hi = arows_v[b, r, hi] + brows_v[b, r, hi]
                        word = rnd16(s_lo) | lax.shift_left(
                            rnd16(s_hi), jnp.uint32(16))
                        obuf_v[b, r, pl.ds(g * 16, 16)] = (
                            lax.bitcast_convert_type(word, jnp.float32))
                    return c2
                lax.fori_loop(0, _KC, addrow, 0, unroll=2)

            @pl.when(s == 0)
            def _add0():
                add_loop(0)

            @pl.when(s == 1)
            def _add1():
                add_loop(1)

            pltpu.async_copy(obuf_v.at[s],
                             out_hbm.at[pl.ds(base + j * _KC, _KC)],
                             sem_w.at[s])
            return carry

        lax.fori_loop(0, nchunks, chunk, 0)
        for jj in (nchunks - 2, nchunks - 1):
            pltpu.make_async_copy(
                obuf_v.at[jj % 2], out_hbm.at[pl.ds(base + jj * _KC, _KC)],
                sem_w.at[jj % 2]).wait()

    return k(A, Bm, row, col)


def _sc_scatter_add(mij, row, n):
    e, f = mij.shape
    e_per_w = e // _NW
    nchunks = e_per_w // _KC
    n_main = 624
    n_tail = n - _NS * n_main
    zrows = 104
    mesh = plsc.VectorSubcoreMesh(core_axis_name="c", subcore_axis_name="s",
                                  num_cores=_NC, num_subcores=_NS)

    @functools.partial(
        pl.kernel, mesh=mesh,
        out_type=jax.ShapeDtypeStruct((_NC * n, f), jnp.float32),
        scratch_types=[
            pltpu.VMEM((e_per_w,), jnp.int32),
            pltpu.VMEM((_KC,), jnp.int32),
            pltpu.VMEM((2, _KC, 128), jnp.float32),
            pltpu.VMEM((zrows, 128), jnp.float32),
            pltpu.VMEM_SHARED((n, 128), jnp.float32),
            pltpu.SemaphoreType.DMA((2,)),
        ],
    )
    def k(mij_hbm, row_hbm, out_hbm, idx_all, idx_sc, rows_v, zero_v,
          agg_sh, sem):
        cid = lax.axis_index("c")
        sid = lax.axis_index("s")
        wid = sid * _NC + cid
        base = wid * e_per_w
        pltpu.sync_copy(row_hbm.at[pl.ds(base, e_per_w)], idx_all)

        def zrow(r, c2):
            def zvec(v, c3):
                zero_v[r, pl.ds(v * 16, 16)] = jnp.zeros((16,), jnp.float32)
                return c3
            return lax.fori_loop(0, 8, zvec, c2)
        lax.fori_loop(0, zrows, zrow, 0)

        def zcp(b, c2):
            pltpu.sync_copy(zero_v,
                            agg_sh.at[pl.ds(sid * n_main + b * zrows, zrows)])
            return c2
        lax.fori_loop(0, n_main // zrows, zcp, 0)

        @pl.when(sid == _NS - 1)
        def _ztail():
            pltpu.sync_copy(zero_v.at[pl.ds(0, n_tail)],
                            agg_sh.at[pl.ds(_NS * n_main, n_tail)])
        plsc.subcore_barrier()

        def fire(j, s):
            pltpu.async_copy(mij_hbm.at[pl.ds(base + j * _KC, _KC)],
                             rows_v.at[s], sem.at[s])

        fire(0, 0)

        def chunk(j, carry):
            s = lax.rem(j, 2)

            @pl.when(j + 1 < nchunks)
            def _prefetch():
                fire(j + 1, 1 - s)

            def icp(v, c2):
                sl = pl.ds(v * 16, 16)
                idx_sc[sl] = idx_all[pl.ds(j * _KC + v * 16, 16)]
                return c2
            lax.fori_loop(0, _KC // 16, icp, 0)

            pltpu.make_async_copy(mij_hbm.at[pl.ds(base + j * _KC, _KC)],
                                  rows_v.at[s], sem.at[s]).wait()
            pltpu.sync_copy(rows_v.at[s], agg_sh.at[idx_sc], add=True)
            return carry
        lax.fori_loop(0, nchunks, chunk, 0)
        plsc.subcore_barrier()

        pltpu.sync_copy(agg_sh.at[pl.ds(sid * n_main, n_main)],
                        out_hbm.at[pl.ds(cid * n + sid * n_main, n_main)])

        @pl.when(sid == _NS - 1)
        def _otail():
            pltpu.sync_copy(agg_sh.at[pl.ds(_NS * n_main, n_tail)],
                            out_hbm.at[pl.ds(cid * n + _NS * n_main, n_tail)])

    res = k(mij, row)
    return res.reshape(_NC, n, f)


def _p2_body(gath_ref, ea_ref, mask_ref, w1e_ref, b1_ref, w2_ref, b2_ref,
             mij_ref):
    u = lax.bitcast_convert_type(gath_ref[...], jnp.uint32)
    f_even = lax.bitcast_convert_type(u << 16, jnp.float32)
    f_odd = lax.bitcast_convert_type(u & jnp.uint32(0xFFFF0000), jnp.float32)
    gath = jnp.concatenate([f_even, f_odd], axis=1)
    pre1 = mask_ref[...] * gath
    pre1 += lax.dot_general(ea_ref[...], w1e_ref[...], (((1,), (0,)), ((), ())),
                            preferred_element_type=jnp.float32)
    pre1 += b1_ref[...]
    h1 = _silu(pre1)
    h2 = lax.dot_general(h1, w2_ref[...], (((1,), (0,)), ((), ())),
                         preferred_element_type=jnp.float32) + b2_ref[...]
    mij_ref[...] = _silu(h2)


def _edge_mlp(gath, ea, mask, We1e, be1, We2, be2):
    e = gath.shape[0]
    blk = 2000
    grid = e // blk
    return pl.pallas_call(
        _p2_body,
        grid=(grid,),
        in_specs=[
            pl.BlockSpec((blk, 64), lambda i: (i, 0)),
            pl.BlockSpec((blk, 16), lambda i: (i, 0)),
            pl.BlockSpec((blk, 1), lambda i: (i, 0)),
            pl.BlockSpec((16, 128), lambda i: (0, 0)),
            pl.BlockSpec((1, 128), lambda i: (0, 0)),
            pl.BlockSpec((128, 128), lambda i: (0, 0)),
            pl.BlockSpec((1, 128), lambda i: (0, 0)),
        ],
        out_specs=pl.BlockSpec((blk, 128), lambda i: (i, 0)),
        out_shape=jax.ShapeDtypeStruct((e, 128), jnp.float32),
    )(gath, ea, mask, We1e, be1.reshape(1, 128), We2, be2.reshape(1, 128))


def _p4_body(h_ref, c_ref, p0_ref, p1_ref, w1b_ref, b1_ref, w2_ref, b2_ref,
             out_ref):
    pre = c_ref[...] + b1_ref[...]
    agg = p0_ref[...] + p1_ref[...]
    pre += lax.dot_general(agg, w1b_ref[...], (((1,), (0,)), ((), ())),
                           preferred_element_type=jnp.float32)
    h1 = _silu(pre)
    o = lax.dot_general(h1, w2_ref[...], (((1,), (0,)), ((), ())),
                        preferred_element_type=jnp.float32) + b2_ref[...]
    out_ref[...] = h_ref[...] + o


def _node_mlp(hf, C, p0, p1, Wn1b, bn1, Wn2, bn2):
    n, f = hf.shape
    blk = 2000
    grid = n // blk
    return pl.pallas_call(
        _p4_body,
        grid=(grid,),
        in_specs=[
            pl.BlockSpec((blk, f), lambda i: (i, 0)),
            pl.BlockSpec((blk, 128), lambda i: (i, 0)),
            pl.BlockSpec((blk, 128), lambda i: (i, 0)),
            pl.BlockSpec((blk, 128), lambda i: (i, 0)),
            pl.BlockSpec((128, 128), lambda i: (0, 0)),
            pl.BlockSpec((1, 128), lambda i: (0, 0)),
            pl.BlockSpec((128, f), lambda i: (0, 0)),
            pl.BlockSpec((1, f), lambda i: (0, 0)),
        ],
        out_specs=pl.BlockSpec((blk, f), lambda i: (i, 0)),
        out_shape=jax.ShapeDtypeStruct((n, f), jnp.float32),
    )(hf, C, p0, p1, Wn1b, bn1.reshape(1, 128), Wn2, bn2.reshape(1, f))


def kernel(h, edge_index, edge_attr, edge_mask, We1, be1, We2, be2,
           Wn1, bn1, Wn2, bn2):
    B, N, F = h.shape
    E = edge_index.shape[1]
    hf = h.reshape(N, F)
    row = edge_index[0, :, 0].astype(jnp.int32)
    col = edge_index[0, :, 1].astype(jnp.int32)
    ea = edge_attr.reshape(E, -1)
    mask = edge_mask.reshape(E, 1)

    Wcat = jnp.concatenate([We1[:F], We1[F:2 * F], Wn1[:F]], axis=1)
    A, Bm, C = _precompute_tables(hf, Wcat)

    gath = _sc_gather_add(A, Bm, row, col)

    k64 = jnp.arange(64)
    perm = jnp.concatenate([32 * (k64 // 16) + k64 % 16,
                            32 * (k64 // 16) + 16 + k64 % 16])
    mij = _edge_mlp(gath, ea, mask, We1[2 * F:][:, perm], be1[perm],
                    We2[perm, :], be2)

    partials = _sc_scatter_add(mij, row, N)

    out = _node_mlp(hf, C, partials[0], partials[1], Wn1[F:], bn1, Wn2, bn2)
    return (out.reshape(B, N, F), mij.reshape(B, E, 128))


# speedup vs baseline: 1.1920x; 1.1226x over previous
import functools

import jax
import jax.numpy as jnp
from jax import lax
from jax.experimental import pallas as pl
from jax.experimental.pallas import tpu as pltpu
from jax.experimental.pallas import tpu_sc as plsc

_NC, _NS = 2, 16
_NW = _NC * _NS
_KC = 80


def _silu(x):
    return x * jax.nn.sigmoid(x)


def _p0_body(h_ref, w_ref, a_ref, b_ref, c_ref):
    hw = lax.dot_general(h_ref[...], w_ref[...], (((1,), (0,)), ((), ())),
                         preferred_element_type=jnp.float32)
    a_ref[...] = hw[:, 0:128]
    b_ref[...] = hw[:, 128:256]
    c_ref[...] = hw[:, 256:384]


def _precompute_tables(hf, Wcat):
    n, f = hf.shape
    blk = 2000
    grid = n // blk
    return pl.pallas_call(
        _p0_body,
        grid=(grid,),
        in_specs=[
            pl.BlockSpec((blk, f), lambda i: (i, 0)),
            pl.BlockSpec((f, 384), lambda i: (0, 0)),
        ],
        out_specs=[
            pl.BlockSpec((blk, 128), lambda i: (i, 0)),
            pl.BlockSpec((blk, 128), lambda i: (i, 0)),
            pl.BlockSpec((blk, 128), lambda i: (i, 0)),
        ],
        out_shape=[jax.ShapeDtypeStruct((n, 128), jnp.float32)] * 3,
    )(hf, Wcat)


def _sc_gather_add(A, Bm, row, col):
    n, f = A.shape
    e = row.shape[0]
    e_per_w = e // _NW
    nchunks = e_per_w // _KC
    mesh = plsc.VectorSubcoreMesh(core_axis_name="c", subcore_axis_name="s",
                                  num_cores=_NC, num_subcores=_NS)

    @functools.partial(
        pl.kernel, mesh=mesh,
        out_type=jax.ShapeDtypeStruct((e, f // 2), jnp.float32),
        scratch_types=[
            pltpu.VMEM((e_per_w,), jnp.int32),
            pltpu.VMEM((e_per_w,), jnp.int32),
            pltpu.VMEM((3, _KC, 128), jnp.float32),
            pltpu.VMEM((3, _KC, 128), jnp.float32),
            pltpu.VMEM((3, _KC, 64), jnp.float32),
            pltpu.SemaphoreType.DMA((3,)),
            pltpu.SemaphoreType.DMA((3,)),
            pltpu.SemaphoreType.DMA((3,)),
        ],
    )
    def k(a_hbm, b_hbm, row_hbm, col_hbm, out_hbm,
          idxr_v, idxc_v, arows_v, brows_v, obuf_v, sem_a, sem_b, sem_w):
        wid = lax.axis_index("s") * _NC + lax.axis_index("c")
        base = wid * e_per_w
        pltpu.sync_copy(row_hbm.at[pl.ds(base, e_per_w)], idxr_v)
        pltpu.sync_copy(col_hbm.at[pl.ds(base, e_per_w)], idxc_v)

        def fire(j, s):
            pltpu.async_copy(a_hbm.at[idxr_v.at[pl.ds(j * _KC, _KC)]],
                             arows_v.at[s], sem_a.at[s])
            pltpu.async_copy(b_hbm.at[idxc_v.at[pl.ds(j * _KC, _KC)]],
                             brows_v.at[s], sem_b.at[s])

        fire(0, 0)
        fire(1, 1)

        def chunk(j, carry):
            s = lax.rem(j, 3)

            @pl.when(j >= 3)
            def _drain_wb():
                pltpu.make_async_copy(
                    obuf_v.at[s], out_hbm.at[pl.ds(base + (j - 3) * _KC, _KC)],
                    sem_w.at[s]).wait()

            @pl.when(j + 2 < nchunks)
            def _prefetch():
                fire(j + 2, lax.rem(j + 2, 3))

            pltpu.make_async_copy(a_hbm.at[idxr_v.at[pl.ds(j * _KC, _KC)]],
                                  arows_v.at[s], sem_a.at[s]).wait()
            pltpu.make_async_copy(b_hbm.at[idxc_v.at[pl.ds(j * _KC, _KC)]],
                                  brows_v.at[s], sem_b.at[s]).wait()

            def rnd16(x):
                u = lax.bitcast_convert_type(x, jnp.uint32)
                rnd = jnp.uint32(0x7FFF) + (
                    lax.shift_right_logical(u, jnp.uint32(16)) & jnp.uint32(1))
                return lax.shift_right_logical(u + rnd, jnp.uint32(16))

            def add_loop(b):
                def addrow(r, c2):
                    for g in range(4):
                        lo = pl.ds(g * 32, 16)
                        hi = pl.ds(g * 32 + 16, 16)
                        s_lo = arows_v[b, r, lo] + brows_v[b, r, lo]
                        s_hi = arows_v[b, r, hi] + brows_v[b, r, hi]
                        word = rnd16(s_lo) | lax.shift_left(
                            rnd16(s_hi), jnp.uint32(16))
                        obuf_v[b, r, pl.ds(g * 16, 16)] = (
                            lax.bitcast_convert_type(word, jnp.float32))
                    return c2
                lax.fori_loop(0, _KC, addrow, 0, unroll=2)

            @pl.when(s == 0)
            def _add0():
                add_loop(0)

            @pl.when(s == 1)
            def _add1():
                add_loop(1)

            @pl.when(s == 2)
            def _add2():
                add_loop(2)

            pltpu.async_copy(obuf_v.at[s],
                             out_hbm.at[pl.ds(base + j * _KC, _KC)],
                             sem_w.at[s])
            return carry

        lax.fori_loop(0, nchunks, chunk, 0)
        for jj in (nchunks - 3, nchunks - 2, nchunks - 1):
            pltpu.make_async_copy(
                obuf_v.at[jj % 3], out_hbm.at[pl.ds(base + jj * _KC, _KC)],
                sem_w.at[jj % 3]).wait()

    return k(A, Bm, row, col)


def _sc_scatter_add(mij, row, n):
    e, f = mij.shape
    e_per_w = e // _NW
    nchunks = e_per_w // _KC
    n_main = 624
    n_tail = n - _NS * n_main
    zrows = 104
    mesh = plsc.VectorSubcoreMesh(core_axis_name="c", subcore_axis_name="s",
                                  num_cores=_NC, num_subcores=_NS)

    @functools.partial(
        pl.kernel, mesh=mesh,
        out_type=jax.ShapeDtypeStruct((_NC * n, f), jnp.float32),
        scratch_types=[
            pltpu.VMEM((e_per_w,), jnp.int32),
            pltpu.VMEM((_KC,), jnp.int32),
            pltpu.VMEM((2, _KC, 128), jnp.float32),
            pltpu.VMEM((zrows, 128), jnp.float32),
            pltpu.VMEM_SHARED((n, 128), jnp.float32),
            pltpu.SemaphoreType.DMA((2,)),
        ],
    )
    def k(mij_hbm, row_hbm, out_hbm, idx_all, idx_sc, rows_v, zero_v,
          agg_sh, sem):
        cid = lax.axis_index("c")
        sid = lax.axis_index("s")
        wid = sid * _NC + cid
        base = wid * e_per_w
        pltpu.sync_copy(row_hbm.at[pl.ds(base, e_per_w)], idx_all)

        def zrow(r, c2):
            def zvec(v, c3):
                zero_v[r, pl.ds(v * 16, 16)] = jnp.zeros((16,), jnp.float32)
                return c3
            return lax.fori_loop(0, 8, zvec, c2)
        lax.fori_loop(0, zrows, zrow, 0)

        def zcp(b, c2):
            pltpu.sync_copy(zero_v,
                            agg_sh.at[pl.ds(sid * n_main + b * zrows, zrows)])
            return c2
        lax.fori_loop(0, n_main // zrows, zcp, 0)

        @pl.when(sid == _NS - 1)
        def _ztail():
            pltpu.sync_copy(zero_v.at[pl.ds(0, n_tail)],
                            agg_sh.at[pl.ds(_NS * n_main, n_tail)])
        plsc.subcore_barrier()

        def fire(j, s):
            pltpu.async_copy(mij_hbm.at[pl.ds(base + j * _KC, _KC)],
                             rows_v.at[s], sem.at[s])

        fire(0, 0)

        def chunk(j, carry):
            s = lax.rem(j, 2)

            @pl.when(j + 1 < nchunks)
            def _prefetch():
                fire(j + 1, 1 - s)

            def icp(v, c2):
                sl = pl.ds(v * 16, 16)
                idx_sc[sl] = idx_all[pl.ds(j * _KC + v * 16, 16)]
                return c2
            lax.fori_loop(0, _KC // 16, icp, 0)

            pltpu.make_async_copy(mij_hbm.at[pl.ds(base + j * _KC, _KC)],
                                  rows_v.at[s], sem.at[s]).wait()
            pltpu.sync_copy(rows_v.at[s], agg_sh.at[idx_sc], add=True)
            return carry
        lax.fori_loop(0, nchunks, chunk, 0)
        plsc.subcore_barrier()

        pltpu.sync_copy(agg_sh.at[pl.ds(sid * n_main, n_main)],
                        out_hbm.at[pl.ds(cid * n + sid * n_main, n_main)])

        @pl.when(sid == _NS - 1)
        def _otail():
            pltpu.sync_copy(agg_sh.at[pl.ds(_NS * n_main, n_tail)],
                            out_hbm.at[pl.ds(cid * n + _NS * n_main, n_tail)])

    res = k(mij, row)
    return res.reshape(_NC, n, f)


def _p2_body(gath_ref, ea_ref, mask_ref, w1e_ref, b1_ref, w2_ref, b2_ref,
             mij_ref):
    u = lax.bitcast_convert_type(gath_ref[...], jnp.uint32)
    f_even = lax.bitcast_convert_type(u << 16, jnp.float32)
    f_odd = lax.bitcast_convert_type(u & jnp.uint32(0xFFFF0000), jnp.float32)
    gath = jnp.concatenate([f_even, f_odd], axis=1)
    pre1 = mask_ref[...] * gath
    pre1 += lax.dot_general(ea_ref[...], w1e_ref[...], (((1,), (0,)), ((), ())),
                            preferred_element_type=jnp.float32)
    pre1 += b1_ref[...]
    h1 = _silu(pre1)
    h2 = lax.dot_general(h1, w2_ref[...], (((1,), (0,)), ((), ())),
                         preferred_element_type=jnp.float32) + b2_ref[...]
    mij_ref[...] = _silu(h2)


def _edge_mlp(gath, ea, mask, We1e, be1, We2, be2):
    e = gath.shape[0]
    blk = 2000
    grid = e // blk
    return pl.pallas_call(
        _p2_body,
        grid=(grid,),
        in_specs=[
            pl.BlockSpec((blk, 64), lambda i: (i, 0)),
            pl.BlockSpec((blk, 16), lambda i: (i, 0)),
            pl.BlockSpec((blk, 1), lambda i: (i, 0)),
            pl.BlockSpec((16, 128), lambda i: (0, 0)),
            pl.BlockSpec((1, 128), lambda i: (0, 0)),
            pl.BlockSpec((128, 128), lambda i: (0, 0)),
            pl.BlockSpec((1, 128), lambda i: (0, 0)),
        ],
        out_specs=pl.BlockSpec((blk, 128), lambda i: (i, 0)),
        out_shape=jax.ShapeDtypeStruct((e, 128), jnp.float32),
    )(gath, ea, mask, We1e, be1.reshape(1, 128), We2, be2.reshape(1, 128))


def _p4_body(h_ref, c_ref, p0_ref, p1_ref, w1b_ref, b1_ref, w2_ref, b2_ref,
             out_ref):
    pre = c_ref[...] + b1_ref[...]
    agg = p0_ref[...] + p1_ref[...]
    pre += lax.dot_general(agg, w1b_ref[...], (((1,), (0,)), ((), ())),
                           preferred_element_type=jnp.float32)
    h1 = _silu(pre)
    o = lax.dot_general(h1, w2_ref[...], (((1,), (0,)), ((), ())),
                        preferred_element_type=jnp.float32) + b2_ref[...]
    out_ref[...] = h_ref[...] + o


def _node_mlp(hf, C, p0, p1, Wn1b, bn1, Wn2, bn2):
    n, f = hf.shape
    blk = 2000
    grid = n // blk
    return pl.pallas_call(
        _p4_body,
        grid=(grid,),
        in_specs=[
            pl.BlockSpec((blk, f), lambda i: (i, 0)),
            pl.BlockSpec((blk, 128), lambda i: (i, 0)),
            pl.BlockSpec((blk, 128), lambda i: (i, 0)),
            pl.BlockSpec((blk, 128), lambda i: (i, 0)),
            pl.BlockSpec((128, 128), lambda i: (0, 0)),
            pl.BlockSpec((1, 128), lambda i: (0, 0)),
            pl.BlockSpec((128, f), lambda i: (0, 0)),
            pl.BlockSpec((1, f), lambda i: (0, 0)),
        ],
        out_specs=pl.BlockSpec((blk, f), lambda i: (i, 0)),
        out_shape=jax.ShapeDtypeStruct((n, f), jnp.float32),
    )(hf, C, p0, p1, Wn1b, bn1.reshape(1, 128), Wn2, bn2.reshape(1, f))


def kernel(h, edge_index, edge_attr, edge_mask, We1, be1, We2, be2,
           Wn1, bn1, Wn2, bn2):
    B, N, F = h.shape
    E = edge_index.shape[1]
    hf = h.reshape(N, F)
    row = edge_index[0, :, 0].astype(jnp.int32)
    col = edge_index[0, :, 1].astype(jnp.int32)
    ea = edge_attr.reshape(E, -1)
    mask = edge_mask.reshape(E, 1)

    Wcat = jnp.concatenate([We1[:F], We1[F:2 * F], Wn1[:F]], axis=1)
    A, Bm, C = _precompute_tables(hf, Wcat)

    gath = _sc_gather_add(A, Bm, row, col)

    k64 = jnp.arange(64)
    perm = jnp.concatenate([32 * (k64 // 16) + k64 % 16,
                            32 * (k64 // 16) + 16 + k64 % 16])
    mij = _edge_mlp(gath, ea, mask, We1[2 * F:][:, perm], be1[perm],
                    We2[perm, :], be2)

    partials = _sc_scatter_add(mij, row, N)

    out = _node_mlp(hf, C, partials[0], partials[1], Wn1[F:], bn1, Wn2, bn2)
    return (out.reshape(B, N, F), mij.reshape(B, E, 128))


# speedup vs baseline: 1.3160x; 1.1041x over previous
import functools

import jax
import jax.numpy as jnp
from jax import lax
from jax.experimental import pallas as pl
from jax.experimental.pallas import tpu as pltpu
from jax.experimental.pallas import tpu_sc as plsc

_NC, _NS = 2, 16
_NW = _NC * _NS
_KC = 80


def _silu(x):
    return x * jax.nn.sigmoid(x)


def _p0_body(h_ref, w_ref, a_ref, b_ref, c_ref):
    hw = lax.dot_general(h_ref[...], w_ref[...], (((1,), (0,)), ((), ())),
                         preferred_element_type=jnp.float32)
    a_ref[...] = hw[:, 0:128]
    b_ref[...] = hw[:, 128:256]
    c_ref[...] = hw[:, 256:384]


def _precompute_tables(hf, Wcat):
    n, f = hf.shape
    blk = 2000
    grid = n // blk
    return pl.pallas_call(
        _p0_body,
        grid=(grid,),
        in_specs=[
            pl.BlockSpec((blk, f), lambda i: (i, 0)),
            pl.BlockSpec((f, 384), lambda i: (0, 0)),
        ],
        out_specs=[
            pl.BlockSpec((blk, 128), lambda i: (i, 0)),
            pl.BlockSpec((blk, 128), lambda i: (i, 0)),
            pl.BlockSpec((blk, 128), lambda i: (i, 0)),
        ],
        out_shape=[jax.ShapeDtypeStruct((n, 128), jnp.float32)] * 3,
    )(hf, Wcat)


def _sc_gather_add(A, Bm, row, col):
    n, f = A.shape
    e = row.shape[0]
    e_per_w = e // _NW
    nchunks = e_per_w // _KC
    mesh = plsc.VectorSubcoreMesh(core_axis_name="c", subcore_axis_name="s",
                                  num_cores=_NC, num_subcores=_NS)

    @functools.partial(
        pl.kernel, mesh=mesh,
        out_type=jax.ShapeDtypeStruct((e, f // 2), jnp.float32),
        scratch_types=[
            pltpu.VMEM((e_per_w,), jnp.int32),
            pltpu.VMEM((e_per_w,), jnp.int32),
            pltpu.VMEM((3, _KC, 128), jnp.float32),
            pltpu.VMEM((3, _KC, 128), jnp.float32),
            pltpu.VMEM((3, _KC, 64), jnp.float32),
            pltpu.SemaphoreType.DMA((3,)),
            pltpu.SemaphoreType.DMA((3,)),
            pltpu.SemaphoreType.DMA((3,)),
        ],
    )
    def k(a_hbm, b_hbm, row_hbm, col_hbm, out_hbm,
          idxr_v, idxc_v, arows_v, brows_v, obuf_v, sem_a, sem_b, sem_w):
        wid = lax.axis_index("s") * _NC + lax.axis_index("c")
        base = wid * e_per_w
        pltpu.sync_copy(row_hbm.at[pl.ds(base, e_per_w)], idxr_v)
        pltpu.sync_copy(col_hbm.at[pl.ds(base, e_per_w)], idxc_v)

        def fire(j, s):
            pltpu.async_copy(a_hbm.at[idxr_v.at[pl.ds(j * _KC, _KC)]],
                             arows_v.at[s], sem_a.at[s])
            pltpu.async_copy(b_hbm.at[idxc_v.at[pl.ds(j * _KC, _KC)]],
                             brows_v.at[s], sem_b.at[s])

        fire(0, 0)
        fire(1, 1)

        def chunk(j, carry):
            s = lax.rem(j, 3)

            @pl.when(j >= 3)
            def _drain_wb():
                pltpu.make_async_copy(
                    obuf_v.at[s], out_hbm.at[pl.ds(base + (j - 3) * _KC, _KC)],
                    sem_w.at[s]).wait()

            @pl.when(j + 2 < nchunks)
            def _prefetch():
                fire(j + 2, lax.rem(j + 2, 3))

            pltpu.make_async_copy(a_hbm.at[idxr_v.at[pl.ds(j * _KC, _KC)]],
                                  arows_v.at[s], sem_a.at[s]).wait()
            pltpu.make_async_copy(b_hbm.at[idxc_v.at[pl.ds(j * _KC, _KC)]],
                                  brows_v.at[s], sem_b.at[s]).wait()

            def rnd16(x):
                u = lax.bitcast_convert_type(x, jnp.uint32)
                rnd = jnp.uint32(0x7FFF) + (
                    lax.shift_right_logical(u, jnp.uint32(16)) & jnp.uint32(1))
                return lax.shift_right_logical(u + rnd, jnp.uint32(16))

            def add_loop(b):
                def addrow(r, c2):
                    for g in range(4):
                        lo = pl.ds(g * 32, 16)
                        hi = pl.ds(g * 32 + 16, 16)
                        s_lo = arows_v[b, r, lo] + brows_v[b, r, lo]
                        s_hi = arows_v[b, r, hi] + brows_v[b, r, hi]
                        word = rnd16(s_lo) | lax.shift_left(
                            rnd16(s_hi), jnp.uint32(16))
                        obuf_v[b, r, pl.ds(g * 16, 16)] = (
                            lax.bitcast_convert_type(word, jnp.float32))
                    return c2
                lax.fori_loop(0, _KC, addrow, 0, unroll=2)

            @pl.when(s == 0)
            def _add0():
                add_loop(0)

            @pl.when(s == 1)
            def _add1():
                add_loop(1)

            @pl.when(s == 2)
            def _add2():
                add_loop(2)

            pltpu.async_copy(obuf_v.at[s],
                             out_hbm.at[pl.ds(base + j * _KC, _KC)],
                             sem_w.at[s])
            return carry

        lax.fori_loop(0, nchunks, chunk, 0)
        for jj in (nchunks - 3, nchunks - 2, nchunks - 1):
            pltpu.make_async_copy(
                obuf_v.at[jj % 3], out_hbm.at[pl.ds(base + jj * _KC, _KC)],
                sem_w.at[jj % 3]).wait()

    return k(A, Bm, row, col)


def _sc_scatter_add(mij, row, n):
    e, f = mij.shape
    e_per_w = e // _NW
    nchunks = e_per_w // _KC
    n_main = 624
    n_tail = n - _NS * n_main
    zrows = 48
    mesh = plsc.VectorSubcoreMesh(core_axis_name="c", subcore_axis_name="s",
                                  num_cores=_NC, num_subcores=_NS)

    @functools.partial(
        pl.kernel, mesh=mesh,
        out_type=jax.ShapeDtypeStruct((_NC * n, f), jnp.float32),
        scratch_types=[
            pltpu.VMEM((e_per_w,), jnp.int32),
            pltpu.VMEM((_KC,), jnp.int32),
            pltpu.VMEM((3, _KC, 128), jnp.float32),
            pltpu.VMEM((zrows, 128), jnp.float32),
            pltpu.VMEM_SHARED((n, 128), jnp.float32),
            pltpu.SemaphoreType.DMA((3,)),
        ],
    )
    def k(mij_hbm, row_hbm, out_hbm, idx_all, idx_sc, rows_v, zero_v,
          agg_sh, sem):
        cid = lax.axis_index("c")
        sid = lax.axis_index("s")
        wid = sid * _NC + cid
        base = wid * e_per_w
        pltpu.sync_copy(row_hbm.at[pl.ds(base, e_per_w)], idx_all)

        def zrow(r, c2):
            def zvec(v, c3):
                zero_v[r, pl.ds(v * 16, 16)] = jnp.zeros((16,), jnp.float32)
                return c3
            return lax.fori_loop(0, 8, zvec, c2)
        lax.fori_loop(0, zrows, zrow, 0)

        def zcp(b, c2):
            pltpu.sync_copy(zero_v,
                            agg_sh.at[pl.ds(sid * n_main + b * zrows, zrows)])
            return c2
        lax.fori_loop(0, n_main // zrows, zcp, 0)

        @pl.when(sid == _NS - 1)
        def _ztail():
            pltpu.sync_copy(zero_v.at[pl.ds(0, n_tail)],
                            agg_sh.at[pl.ds(_NS * n_main, n_tail)])
        plsc.subcore_barrier()

        def fire(j, s):
            pltpu.async_copy(mij_hbm.at[pl.ds(base + j * _KC, _KC)],
                             rows_v.at[s], sem.at[s])

        fire(0, 0)
        fire(1, 1)

        def chunk(j, carry):
            s = lax.rem(j, 3)

            @pl.when(j + 2 < nchunks)
            def _prefetch():
                fire(j + 2, lax.rem(j + 2, 3))

            def icp(v, c2):
                sl = pl.ds(v * 16, 16)
                idx_sc[sl] = idx_all[pl.ds(j * _KC + v * 16, 16)]
                return c2
            lax.fori_loop(0, _KC // 16, icp, 0)

            pltpu.make_async_copy(mij_hbm.at[pl.ds(base + j * _KC, _KC)],
                                  rows_v.at[s], sem.at[s]).wait()
            pltpu.sync_copy(rows_v.at[s], agg_sh.at[idx_sc], add=True)
            return carry
        lax.fori_loop(0, nchunks, chunk, 0)
        plsc.subcore_barrier()

        pltpu.sync_copy(agg_sh.at[pl.ds(sid * n_main, n_main)],
                        out_hbm.at[pl.ds(cid * n + sid * n_main, n_main)])

        @pl.when(sid == _NS - 1)
        def _otail():
            pltpu.sync_copy(agg_sh.at[pl.ds(_NS * n_main, n_tail)],
                            out_hbm.at[pl.ds(cid * n + _NS * n_main, n_tail)])

    res = k(mij, row)
    return res.reshape(_NC, n, f)


def _p2_body(gath_ref, ea_ref, mask_ref, w1e_ref, b1_ref, w2_ref, b2_ref,
             mij_ref):
    u = lax.bitcast_convert_type(gath_ref[...], jnp.uint32)
    f_even = lax.bitcast_convert_type(u << 16, jnp.float32)
    f_odd = lax.bitcast_convert_type(u & jnp.uint32(0xFFFF0000), jnp.float32)
    gath = jnp.concatenate([f_even, f_odd], axis=1)
    pre1 = mask_ref[...] * gath
    pre1 += lax.dot_general(ea_ref[...], w1e_ref[...], (((1,), (0,)), ((), ())),
                            preferred_element_type=jnp.float32)
    pre1 += b1_ref[...]
    h1 = _silu(pre1)
    h2 = lax.dot_general(h1, w2_ref[...], (((1,), (0,)), ((), ())),
                         preferred_element_type=jnp.float32) + b2_ref[...]
    mij_ref[...] = _silu(h2)


def _edge_mlp(gath, ea, mask, We1e, be1, We2, be2):
    e = gath.shape[0]
    blk = 4000
    grid = e // blk
    return pl.pallas_call(
        _p2_body,
        grid=(grid,),
        in_specs=[
            pl.BlockSpec((blk, 64), lambda i: (i, 0)),
            pl.BlockSpec((blk, 16), lambda i: (i, 0)),
            pl.BlockSpec((blk, 1), lambda i: (i, 0)),
            pl.BlockSpec((16, 128), lambda i: (0, 0)),
            pl.BlockSpec((1, 128), lambda i: (0, 0)),
            pl.BlockSpec((128, 128), lambda i: (0, 0)),
            pl.BlockSpec((1, 128), lambda i: (0, 0)),
        ],
        out_specs=pl.BlockSpec((blk, 128), lambda i: (i, 0)),
        out_shape=jax.ShapeDtypeStruct((e, 128), jnp.float32),
    )(gath, ea, mask, We1e, be1.reshape(1, 128), We2, be2.reshape(1, 128))


def _p4_body(h_ref, c_ref, p0_ref, p1_ref, w1b_ref, b1_ref, w2_ref, b2_ref,
             out_ref):
    pre = c_ref[...] + b1_ref[...]
    agg = p0_ref[...] + p1_ref[...]
    pre += lax.dot_general(agg, w1b_ref[...], (((1,), (0,)), ((), ())),
                           preferred_element_type=jnp.float32)
    h1 = _silu(pre)
    o = lax.dot_general(h1, w2_ref[...], (((1,), (0,)), ((), ())),
                        preferred_element_type=jnp.float32) + b2_ref[...]
    out_ref[...] = h_ref[...] + o


def _node_mlp(hf, C, p0, p1, Wn1b, bn1, Wn2, bn2):
    n, f = hf.shape
    blk = 2000
    grid = n // blk
    return pl.pallas_call(
        _p4_body,
        grid=(grid,),
        in_specs=[
            pl.BlockSpec((blk, f), lambda i: (i, 0)),
            pl.BlockSpec((blk, 128), lambda i: (i, 0)),
            pl.BlockSpec((blk, 128), lambda i: (i, 0)),
            pl.BlockSpec((blk, 128), lambda i: (i, 0)),
            pl.BlockSpec((128, 128), lambda i: (0, 0)),
            pl.BlockSpec((1, 128), lambda i: (0, 0)),
            pl.BlockSpec((128, f), lambda i: (0, 0)),
            pl.BlockSpec((1, f), lambda i: (0, 0)),
        ],
        out_specs=pl.BlockSpec((blk, f), lambda i: (i, 0)),
        out_shape=jax.ShapeDtypeStruct((n, f), jnp.float32),
    )(hf, C, p0, p1, Wn1b, bn1.reshape(1, 128), Wn2, bn2.reshape(1, f))


def kernel(h, edge_index, edge_attr, edge_mask, We1, be1, We2, be2,
           Wn1, bn1, Wn2, bn2):
    B, N, F = h.shape
    E = edge_index.shape[1]
    hf = h.reshape(N, F)
    row = edge_index[0, :, 0].astype(jnp.int32)
    col = edge_index[0, :, 1].astype(jnp.int32)
    ea = edge_attr.reshape(E, -1)
    mask = edge_mask.reshape(E, 1)

    Wcat = jnp.concatenate([We1[:F], We1[F:2 * F], Wn1[:F]], axis=1)
    A, Bm, C = _precompute_tables(hf, Wcat)

    gath = _sc_gather_add(A, Bm, row, col)

    k64 = jnp.arange(64)
    perm = jnp.concatenate([32 * (k64 // 16) + k64 % 16,
                            32 * (k64 // 16) + 16 + k64 % 16])
    mij = _edge_mlp(gath, ea, mask, We1[2 * F:][:, perm], be1[perm],
                    We2[perm, :], be2)

    partials = _sc_scatter_add(mij, row, N)

    out = _node_mlp(hf, C, partials[0], partials[1], Wn1[F:], bn1, Wn2, bn2)
    return (out.reshape(B, N, F), mij.reshape(B, E, 128))
